# baseline (speedup 1.0000x reference)
import functools

import jax
import jax.numpy as jnp
from jax import lax
from jax.experimental import pallas as pl
from jax.experimental.pallas import tpu as pltpu
from jax.experimental.pallas import tpu_sc as plsc

N = 10000
F_IN = 128
H = 128
G = 64
E = 320000

N_PAD = 10112
NROW = N_PAD // 128
NC = 2
NS = 16
NW = NC * NS
CH = 64
NCH = 162
NCHI = NCH + 1
EW = NCH * CH
E_PAD = NW * EW
RPT = N_PAD // NS

_f32 = jnp.float32


_mesh = plsc.VectorSubcoreMesh(core_axis_name="c", subcore_axis_name="s",
                               num_cores=NC, num_subcores=NS)


@functools.partial(
    pl.kernel,
    out_type=(
        jax.ShapeDtypeStruct((NC, N_PAD, H), _f32),
        jax.ShapeDtypeStruct((NW, NROW, 128), _f32),
    ),
    mesh=_mesh,
    compiler_params=pltpu.CompilerParams(needs_layout_passes=False),
    scratch_types=[
        pltpu.VMEM((4, CH), jnp.int32),
        pltpu.VMEM((NROW, 128), _f32),
        pltpu.VMEM((NROW, 128), _f32),
        pltpu.VMEM((NROW, 128), _f32),
        pltpu.VMEM((2, CH), _f32),
        pltpu.VMEM((2 * CH, H), _f32),
        pltpu.VMEM_SHARED((N_PAD, H), _f32),
        pltpu.SemaphoreType.DMA,
        pltpu.SemaphoreType.DMA,
    ],
)
def _edge_kernel(src_hbm, dst_hbm, h_hbm, as_hbm, ad_hbm, acc_out, s_out,
                 sd, as_loc, ad_loc, s_part, ex_buf, rows, acc, sem0, sem1):
    cid = lax.axis_index("c")
    sid = lax.axis_index("s")
    wid = cid * NS + sid
    zero16 = jnp.zeros((16,), _f32)
    z16 = jnp.zeros((16,), jnp.int32)

    def _zrow(i, _):
        for v in range(H // 16):
            rows[i, pl.ds(v * 16, 16)] = zero16
        return 0
    lax.fori_loop(0, 2 * CH, _zrow, 0)
    base = sid * RPT
    for k in range(RPT // (2 * CH)):
        pltpu.sync_copy(rows, acc.at[pl.ds(base + k * 2 * CH, 2 * CH)])
    rem = RPT - (RPT // (2 * CH)) * 2 * CH
    if rem:
        pltpu.sync_copy(rows.at[pl.ds(0, rem)],
                        acc.at[pl.ds(base + (RPT // (2 * CH)) * 2 * CH, rem)])

    def _zs(i, _):
        for v in range(128 // 16):
            s_part[i, pl.ds(v * 16, 16)] = zero16
        return 0
    lax.fori_loop(0, NROW, _zs, 0)

    pltpu.sync_copy(as_hbm, as_loc)
    pltpu.sync_copy(ad_hbm, ad_loc)

    plsc.subcore_barrier()

    def _phase(b, c):
        rb = rows.at[pl.ds(b * CH, CH)]
        sem = sem0 if b == 0 else sem1
        nsem = sem1 if b == 0 else sem0
        pltpu.make_async_copy(h_hbm.at[sd.at[2 * b]], rb, sem).wait()
        nb = 1 - b
        pltpu.sync_copy(src_hbm.at[wid, c + 1], sd.at[2 * nb])
        pltpu.sync_copy(dst_hbm.at[wid, c + 1], sd.at[2 * nb + 1])
        pltpu.async_copy(h_hbm.at[sd.at[2 * nb]],
                         rows.at[pl.ds(nb * CH, CH)], nsem)
        for k in range(CH // 16):
            s16 = sd[2 * b, pl.ds(k * 16, 16)]
            d16 = sd[2 * b + 1, pl.ds(k * 16, 16)]
            e = (plsc.load_gather(as_loc, [s16 >> 7, s16 & 127])
                 + plsc.load_gather(ad_loc, [d16 >> 7, d16 & 127]))
            e = jnp.where(e > 0.0, e, 0.2 * e)
            ex = jnp.exp(e)
            ex_buf[b, pl.ds(k * 16, 16)] = ex
            plsc.addupdate_scatter(s_part, [d16 >> 7, d16 & 127], ex)

        def _mul(i, _):
            for u in range(2):
                a = plsc.load_gather(ex_buf, [z16 + b, z16 + 2 * i + u])
                r = b * CH + 2 * i + u
                for v in range(H // 16):
                    rows[r, pl.ds(v * 16, 16)] = rows[r, pl.ds(v * 16, 16)] * a
            return 0
        lax.fori_loop(0, CH // 2, _mul, 0)

        pltpu.sync_copy(rb, acc.at[sd.at[2 * b + 1]], add=True)

    pltpu.sync_copy(src_hbm.at[wid, 0], sd.at[0])
    pltpu.sync_copy(dst_hbm.at[wid, 0], sd.at[1])
    pltpu.async_copy(h_hbm.at[sd.at[0]], rows.at[pl.ds(0, CH)], sem0)

    def _pair(i, _):
        _phase(0, 2 * i)
        _phase(1, 2 * i + 1)
        return 0
    lax.fori_loop(0, NCH // 2, _pair, 0)

    pltpu.make_async_copy(h_hbm.at[sd.at[0]], rows.at[pl.ds(0, CH)],
                          sem0).wait()

    plsc.subcore_barrier()
    pltpu.sync_copy(s_part, s_out.at[wid])
    pltpu.sync_copy(acc.at[pl.ds(base, RPT)],
                    acc_out.at[cid, pl.ds(base, RPT)])


def _proj_body(x_ref, w_ref, aws_ref, awd_ref, h_ref, as_ref, ad_ref):
    h = jnp.dot(x_ref[...], w_ref[...], preferred_element_type=_f32)
    h_ref[...] = h
    as_ref[...] = jnp.dot(h, aws_ref[...], preferred_element_type=_f32)
    ad_ref[...] = jnp.dot(h, awd_ref[...], preferred_element_type=_f32)


_proj = pl.pallas_call(
    _proj_body,
    out_shape=(
        jax.ShapeDtypeStruct((N_PAD, H), _f32),
        jax.ShapeDtypeStruct((N_PAD, 1), _f32),
        jax.ShapeDtypeStruct((N_PAD, 1), _f32),
    ),
)


def _spsum_body(sp_ref, out_ref):
    s = sp_ref[0]
    for k in range(1, NW):
        s = s + sp_ref[k]
    out_ref[...] = jnp.where(s == 0.0, 1.0, s)


_spsum = pl.pallas_call(
    _spsum_body,
    out_shape=jax.ShapeDtypeStruct((NROW, 128), _f32),
)


def _sum_parts(acc_ref, s_ref, b_ref):
    return (acc_ref[0] + acc_ref[1]) / s_ref[...] + b_ref[...]


def _combine_body(acc_ref, s_ref, b_ref, w_ref, aws_ref, awd_ref,
                  h_ref, as_ref, ad_ref):
    o = jnp.maximum(_sum_parts(acc_ref, s_ref, b_ref), 0.0)
    h = jnp.dot(o, w_ref[...], preferred_element_type=_f32)
    h_ref[...] = h
    as_ref[...] = jnp.dot(h, aws_ref[...], preferred_element_type=_f32)
    ad_ref[...] = jnp.dot(h, awd_ref[...], preferred_element_type=_f32)


_combine = pl.pallas_call(
    _combine_body,
    out_shape=(
        jax.ShapeDtypeStruct((N_PAD, H), _f32),
        jax.ShapeDtypeStruct((N_PAD, 1), _f32),
        jax.ShapeDtypeStruct((N_PAD, 1), _f32),
    ),
)


def _final_body(acc_ref, s_ref, b_ref, bc_ref, br_ref, gw_ref, gb_ref,
                lw_ref, lb_ref, out_ref):
    h2 = _sum_parts(acc_ref, s_ref, b_ref)[:N]
    gate = jnp.dot(h2, gw_ref[...], preferred_element_type=_f32) + gb_ref[...]
    oh = lax.broadcasted_iota(jnp.int32, (N, G), 1) == bc_ref[...]
    gm = jnp.max(jnp.where(oh, gate, -1e30), axis=0, keepdims=True)
    gm_n = jnp.max(jnp.where(oh, gm, -1e30), axis=1, keepdims=True)
    ge = jnp.exp(gate - gm_n)
    gs = jnp.sum(jnp.where(oh, ge, 0.0), axis=0, keepdims=True)
    gs_n = jnp.sum(jnp.where(oh, gs, 0.0), axis=1, keepdims=True)
    wh = h2 * (ge / gs_n)
    oht = (lax.broadcasted_iota(jnp.int32, (G, N), 0) == br_ref[...]).astype(_f32)
    pooled = jnp.dot(oht, wh, preferred_element_type=_f32)
    out_ref[...] = jnp.dot(pooled, lw_ref[...],
                           preferred_element_type=_f32) + lb_ref[...]


_final = pl.pallas_call(
    _final_body,
    out_shape=jax.ShapeDtypeStruct((G, H), _f32),
)


def kernel(x, edge_index, batch, W1, a_src1, a_dst1, b1, W2, a_src2, a_dst2,
           b2, gate_W, gate_b, lin_W, lin_b):
    x_pad = jnp.pad(x, ((0, N_PAD - N), (0, 0)))
    loop = jnp.arange(N, dtype=jnp.int32)
    pad_idx = jnp.full((E_PAD - E - N,), N, jnp.int32)
    dummy = jnp.full((NW, 1, CH), N, jnp.int32)
    src = jnp.concatenate(
        [jnp.concatenate([edge_index[0], loop, pad_idx]).reshape(NW, NCH, CH),
         dummy], axis=1)
    dst = jnp.concatenate(
        [jnp.concatenate([edge_index[1], loop, pad_idx]).reshape(NW, NCH, CH),
         dummy], axis=1)

    h1, as1, ad1 = _proj(x_pad, W1, a_src1.reshape(H, 1), a_dst1.reshape(H, 1))
    acc1, sp1 = _edge_kernel(src, dst, h1, as1.reshape(NROW, 128),
                             ad1.reshape(NROW, 128))
    s1 = _spsum(sp1).reshape(N_PAD, 1)
    h2, as2, ad2 = _combine(acc1, s1, b1.reshape(1, H),
                            W2, a_src2.reshape(H, 1), a_dst2.reshape(H, 1))
    acc2, sp2 = _edge_kernel(src, dst, h2, as2.reshape(NROW, 128),
                             ad2.reshape(NROW, 128))
    s2 = _spsum(sp2).reshape(N_PAD, 1)
    return _final(acc2, s2, b2.reshape(1, H),
                  batch.reshape(N, 1), batch.reshape(1, N),
                  gate_W, gate_b.reshape(1, 1), lin_W, lin_b.reshape(1, H))


# speedup vs baseline: 1.0594x; 1.0594x over previous
import functools

import jax
import jax.numpy as jnp
from jax import lax
from jax.experimental import pallas as pl
from jax.experimental.pallas import tpu as pltpu
from jax.experimental.pallas import tpu_sc as plsc

N = 10000
F_IN = 128
H = 128
G = 64
E = 320000

N_PAD = 10112
NROW = N_PAD // 128
NC = 2
NS = 16
NW = NC * NS
CH = 128
NCH = 81
EW = NCH * CH
E_PAD = NW * EW
RPT = N_PAD // NS

_f32 = jnp.float32


_mesh = plsc.VectorSubcoreMesh(core_axis_name="c", subcore_axis_name="s",
                               num_cores=NC, num_subcores=NS)


@functools.partial(
    pl.kernel,
    out_type=(
        jax.ShapeDtypeStruct((NC, N_PAD, H), _f32),
        jax.ShapeDtypeStruct((NW, NROW, 128), _f32),
    ),
    mesh=_mesh,
    compiler_params=pltpu.CompilerParams(needs_layout_passes=False),
    scratch_types=[
        pltpu.VMEM((2, CH), jnp.int32),
        pltpu.VMEM((NROW, 128), _f32),
        pltpu.VMEM((NROW, 128), _f32),
        pltpu.VMEM((NROW, 128), _f32),
        pltpu.VMEM((2, CH), _f32),
        pltpu.VMEM((CH, H), _f32),
        pltpu.VMEM_SHARED((N_PAD, H), _f32),
    ],
)
def _edge_kernel(src_hbm, dst_hbm, h_hbm, as_hbm, ad_hbm, acc_out, s_out,
                 sd, as_loc, ad_loc, s_part, ex_buf, rows, acc):
    cid = lax.axis_index("c")
    sid = lax.axis_index("s")
    wid = cid * NS + sid
    zero16 = jnp.zeros((16,), _f32)
    z16 = jnp.zeros((16,), jnp.int32)

    def _zrow(i, _):
        for v in range(H // 16):
            rows[i, pl.ds(v * 16, 16)] = zero16
        return 0
    lax.fori_loop(0, CH, _zrow, 0)
    base = sid * RPT
    for k in range(RPT // CH):
        pltpu.sync_copy(rows, acc.at[pl.ds(base + k * CH, CH)])
    rem = RPT - (RPT // CH) * CH
    if rem:
        pltpu.sync_copy(rows.at[pl.ds(0, rem)],
                        acc.at[pl.ds(base + (RPT // CH) * CH, rem)])

    def _zs(i, _):
        for v in range(128 // 16):
            s_part[i, pl.ds(v * 16, 16)] = zero16
        return 0
    lax.fori_loop(0, NROW, _zs, 0)

    pltpu.sync_copy(as_hbm, as_loc)
    pltpu.sync_copy(ad_hbm, ad_loc)

    plsc.subcore_barrier()

    def _chunk(c, _):
        pltpu.sync_copy(src_hbm.at[wid, c], sd.at[0])
        pltpu.sync_copy(dst_hbm.at[wid, c], sd.at[1])
        pltpu.sync_copy(h_hbm.at[sd.at[0]], rows)
        for k in range(CH // 16):
            s16 = sd[0, pl.ds(k * 16, 16)]
            d16 = sd[1, pl.ds(k * 16, 16)]
            e = (plsc.load_gather(as_loc, [s16 >> 7, s16 & 127])
                 + plsc.load_gather(ad_loc, [d16 >> 7, d16 & 127]))
            e = jnp.where(e > 0.0, e, 0.2 * e)
            ex = jnp.exp(e)
            ex_buf[0, pl.ds(k * 16, 16)] = ex
            plsc.addupdate_scatter(s_part, [d16 >> 7, d16 & 127], ex)

        def _mul(i, _):
            for u in range(2):
                a = plsc.load_gather(ex_buf, [z16, z16 + 2 * i + u])
                r = 2 * i + u
                for v in range(H // 16):
                    rows[r, pl.ds(v * 16, 16)] = rows[r, pl.ds(v * 16, 16)] * a
            return 0
        lax.fori_loop(0, CH // 2, _mul, 0)

        pltpu.sync_copy(rows, acc.at[sd.at[1]], add=True)
        return 0

    lax.fori_loop(0, NCH, _chunk, 0)

    plsc.subcore_barrier()
    pltpu.sync_copy(s_part, s_out.at[wid])
    pltpu.sync_copy(acc.at[pl.ds(base, RPT)],
                    acc_out.at[cid, pl.ds(base, RPT)])


def _proj_body(x_ref, w_ref, aws_ref, awd_ref, h_ref, as_ref, ad_ref):
    h = jnp.dot(x_ref[...], w_ref[...], preferred_element_type=_f32)
    h_ref[...] = h
    as_ref[...] = jnp.dot(h, aws_ref[...], preferred_element_type=_f32)
    ad_ref[...] = jnp.dot(h, awd_ref[...], preferred_element_type=_f32)


_proj = pl.pallas_call(
    _proj_body,
    out_shape=(
        jax.ShapeDtypeStruct((N_PAD, H), _f32),
        jax.ShapeDtypeStruct((N_PAD, 1), _f32),
        jax.ShapeDtypeStruct((N_PAD, 1), _f32),
    ),
)


def _spsum_body(sp_ref, out_ref):
    s = sp_ref[0]
    for k in range(1, NW):
        s = s + sp_ref[k]
    out_ref[...] = jnp.where(s == 0.0, 1.0, s)


_spsum = pl.pallas_call(
    _spsum_body,
    out_shape=jax.ShapeDtypeStruct((NROW, 128), _f32),
)


def _sum_parts(acc_ref, s_ref, b_ref):
    return (acc_ref[0] + acc_ref[1]) / s_ref[...] + b_ref[...]


def _combine_body(acc_ref, s_ref, b_ref, w_ref, aws_ref, awd_ref,
                  h_ref, as_ref, ad_ref):
    o = jnp.maximum(_sum_parts(acc_ref, s_ref, b_ref), 0.0)
    h = jnp.dot(o, w_ref[...], preferred_element_type=_f32)
    h_ref[...] = h
    as_ref[...] = jnp.dot(h, aws_ref[...], preferred_element_type=_f32)
    ad_ref[...] = jnp.dot(h, awd_ref[...], preferred_element_type=_f32)


_combine = pl.pallas_call(
    _combine_body,
    out_shape=(
        jax.ShapeDtypeStruct((N_PAD, H), _f32),
        jax.ShapeDtypeStruct((N_PAD, 1), _f32),
        jax.ShapeDtypeStruct((N_PAD, 1), _f32),
    ),
)


def _final_body(acc_ref, s_ref, b_ref, bc_ref, br_ref, gw_ref, gb_ref,
                lw_ref, lb_ref, out_ref):
    h2 = _sum_parts(acc_ref, s_ref, b_ref)[:N]
    gate = jnp.dot(h2, gw_ref[...], preferred_element_type=_f32) + gb_ref[...]
    oh = lax.broadcasted_iota(jnp.int32, (N, G), 1) == bc_ref[...]
    gm = jnp.max(jnp.where(oh, gate, -1e30), axis=0, keepdims=True)
    gm_n = jnp.max(jnp.where(oh, gm, -1e30), axis=1, keepdims=True)
    ge = jnp.exp(gate - gm_n)
    gs = jnp.sum(jnp.where(oh, ge, 0.0), axis=0, keepdims=True)
    gs_n = jnp.sum(jnp.where(oh, gs, 0.0), axis=1, keepdims=True)
    wh = h2 * (ge / gs_n)
    oht = (lax.broadcasted_iota(jnp.int32, (G, N), 0) == br_ref[...]).astype(_f32)
    pooled = jnp.dot(oht, wh, preferred_element_type=_f32)
    out_ref[...] = jnp.dot(pooled, lw_ref[...],
                           preferred_element_type=_f32) + lb_ref[...]


_final = pl.pallas_call(
    _final_body,
    out_shape=jax.ShapeDtypeStruct((G, H), _f32),
)


def kernel(x, edge_index, batch, W1, a_src1, a_dst1, b1, W2, a_src2, a_dst2,
           b2, gate_W, gate_b, lin_W, lin_b):
    x_pad = jnp.pad(x, ((0, N_PAD - N), (0, 0)))
    loop = jnp.arange(N, dtype=jnp.int32)
    pad_idx = jnp.full((E_PAD - E - N,), N, jnp.int32)
    src = jnp.concatenate([edge_index[0], loop, pad_idx]).reshape(NW, NCH, CH)
    dst = jnp.concatenate([edge_index[1], loop, pad_idx]).reshape(NW, NCH, CH)

    h1, as1, ad1 = _proj(x_pad, W1, a_src1.reshape(H, 1), a_dst1.reshape(H, 1))
    acc1, sp1 = _edge_kernel(src, dst, h1, as1.reshape(NROW, 128),
                             ad1.reshape(NROW, 128))
    s1 = _spsum(sp1).reshape(N_PAD, 1)
    h2, as2, ad2 = _combine(acc1, s1, b1.reshape(1, H),
                            W2, a_src2.reshape(H, 1), a_dst2.reshape(H, 1))
    acc2, sp2 = _edge_kernel(src, dst, h2, as2.reshape(NROW, 128),
                             ad2.reshape(NROW, 128))
    s2 = _spsum(sp2).reshape(N_PAD, 1)
    return _final(acc2, s2, b2.reshape(1, H),
                  batch.reshape(N, 1), batch.reshape(1, N),
                  gate_W, gate_b.reshape(1, 1), lin_W, lin_b.reshape(1, H))


# speedup vs baseline: 1.0891x; 1.0281x over previous
import functools

import jax
import jax.numpy as jnp
from jax import lax
from jax.experimental import pallas as pl
from jax.experimental.pallas import tpu as pltpu
from jax.experimental.pallas import tpu_sc as plsc

N = 10000
F_IN = 128
H = 128
G = 64
E = 320000

N_PAD = 10112
NROW = N_PAD // 128
NC = 2
NS = 16
NW = NC * NS
CH = 128
NCH = 81
EW = NCH * CH
E_PAD = NW * EW
RPT = N_PAD // NS

_f32 = jnp.float32


_mesh = plsc.VectorSubcoreMesh(core_axis_name="c", subcore_axis_name="s",
                               num_cores=NC, num_subcores=NS)


@functools.partial(
    pl.kernel,
    out_type=(
        jax.ShapeDtypeStruct((NC, N_PAD, H), _f32),
        jax.ShapeDtypeStruct((NW, NROW, 128), _f32),
    ),
    mesh=_mesh,
    compiler_params=pltpu.CompilerParams(needs_layout_passes=False),
    scratch_types=[
        pltpu.VMEM((2, CH), jnp.int32),
        pltpu.VMEM((NROW, 128), _f32),
        pltpu.VMEM((NROW, 128), _f32),
        pltpu.VMEM((NROW, 128), _f32),
        pltpu.VMEM((2, CH), _f32),
        pltpu.VMEM((CH, H), _f32),
        pltpu.VMEM_SHARED((N_PAD, H), _f32),
        pltpu.SemaphoreType.DMA,
    ],
)
def _edge_kernel(src_hbm, dst_hbm, h_hbm, as_hbm, ad_hbm, acc_out, s_out,
                 sd, as_loc, ad_loc, s_part, ex_buf, rows, acc, sem):
    cid = lax.axis_index("c")
    sid = lax.axis_index("s")
    wid = cid * NS + sid
    zero16 = jnp.zeros((16,), _f32)
    z16 = jnp.zeros((16,), jnp.int32)

    def _zrow(i, _):
        for v in range(H // 16):
            rows[i, pl.ds(v * 16, 16)] = zero16
        return 0
    lax.fori_loop(0, CH, _zrow, 0)
    base = sid * RPT
    for k in range(RPT // CH):
        pltpu.sync_copy(rows, acc.at[pl.ds(base + k * CH, CH)])
    rem = RPT - (RPT // CH) * CH
    if rem:
        pltpu.sync_copy(rows.at[pl.ds(0, rem)],
                        acc.at[pl.ds(base + (RPT // CH) * CH, rem)])

    def _zs(i, _):
        for v in range(128 // 16):
            s_part[i, pl.ds(v * 16, 16)] = zero16
        return 0
    lax.fori_loop(0, NROW, _zs, 0)

    pltpu.sync_copy(as_hbm, as_loc)
    pltpu.sync_copy(ad_hbm, ad_loc)

    plsc.subcore_barrier()

    def _chunk(c, _):
        pltpu.sync_copy(src_hbm.at[wid, c], sd.at[0])
        pltpu.sync_copy(dst_hbm.at[wid, c], sd.at[1])
        pltpu.async_copy(h_hbm.at[sd.at[0]], rows, sem)
        for k in range(CH // 16):
            s16 = sd[0, pl.ds(k * 16, 16)]
            d16 = sd[1, pl.ds(k * 16, 16)]
            e = (plsc.load_gather(as_loc, [s16 >> 7, s16 & 127])
                 + plsc.load_gather(ad_loc, [d16 >> 7, d16 & 127]))
            e = jnp.where(e > 0.0, e, 0.2 * e)
            ex = jnp.exp(e)
            ex_buf[0, pl.ds(k * 16, 16)] = ex
            plsc.addupdate_scatter(s_part, [d16 >> 7, d16 & 127], ex)

        pltpu.make_async_copy(h_hbm.at[sd.at[0]], rows, sem).wait()

        def _mul(i, _):
            for u in range(2):
                a = plsc.load_gather(ex_buf, [z16, z16 + 2 * i + u])
                r = 2 * i + u
                for v in range(H // 16):
                    rows[r, pl.ds(v * 16, 16)] = rows[r, pl.ds(v * 16, 16)] * a
            return 0
        lax.fori_loop(0, CH // 2, _mul, 0)

        pltpu.sync_copy(rows, acc.at[sd.at[1]], add=True)
        return 0

    lax.fori_loop(0, NCH, _chunk, 0)

    plsc.subcore_barrier()
    pltpu.sync_copy(s_part, s_out.at[wid])
    pltpu.sync_copy(acc.at[pl.ds(base, RPT)],
                    acc_out.at[cid, pl.ds(base, RPT)])


def _proj_body(x_ref, w_ref, aws_ref, awd_ref, h_ref, as_ref, ad_ref):
    h = jnp.dot(x_ref[...], w_ref[...], preferred_element_type=_f32)
    h_ref[...] = h
    as_ref[...] = jnp.dot(h, aws_ref[...], preferred_element_type=_f32)
    ad_ref[...] = jnp.dot(h, awd_ref[...], preferred_element_type=_f32)


_proj = pl.pallas_call(
    _proj_body,
    out_shape=(
        jax.ShapeDtypeStruct((N_PAD, H), _f32),
        jax.ShapeDtypeStruct((N_PAD, 1), _f32),
        jax.ShapeDtypeStruct((N_PAD, 1), _f32),
    ),
)


def _spsum_body(sp_ref, out_ref):
    s = sp_ref[0]
    for k in range(1, NW):
        s = s + sp_ref[k]
    out_ref[...] = jnp.where(s == 0.0, 1.0, s)


_spsum = pl.pallas_call(
    _spsum_body,
    out_shape=jax.ShapeDtypeStruct((NROW, 128), _f32),
)


def _sum_parts(acc_ref, s_ref, b_ref):
    return (acc_ref[0] + acc_ref[1]) / s_ref[...] + b_ref[...]


def _combine_body(acc_ref, s_ref, b_ref, w_ref, aws_ref, awd_ref,
                  h_ref, as_ref, ad_ref):
    o = jnp.maximum(_sum_parts(acc_ref, s_ref, b_ref), 0.0)
    h = jnp.dot(o, w_ref[...], preferred_element_type=_f32)
    h_ref[...] = h
    as_ref[...] = jnp.dot(h, aws_ref[...], preferred_element_type=_f32)
    ad_ref[...] = jnp.dot(h, awd_ref[...], preferred_element_type=_f32)


_combine = pl.pallas_call(
    _combine_body,
    out_shape=(
        jax.ShapeDtypeStruct((N_PAD, H), _f32),
        jax.ShapeDtypeStruct((N_PAD, 1), _f32),
        jax.ShapeDtypeStruct((N_PAD, 1), _f32),
    ),
)


def _final_body(acc_ref, s_ref, b_ref, bc_ref, br_ref, gw_ref, gb_ref,
                lw_ref, lb_ref, out_ref):
    h2 = _sum_parts(acc_ref, s_ref, b_ref)[:N]
    gate = jnp.dot(h2, gw_ref[...], preferred_element_type=_f32) + gb_ref[...]
    oh = lax.broadcasted_iota(jnp.int32, (N, G), 1) == bc_ref[...]
    gm = jnp.max(jnp.where(oh, gate, -1e30), axis=0, keepdims=True)
    gm_n = jnp.max(jnp.where(oh, gm, -1e30), axis=1, keepdims=True)
    ge = jnp.exp(gate - gm_n)
    gs = jnp.sum(jnp.where(oh, ge, 0.0), axis=0, keepdims=True)
    gs_n = jnp.sum(jnp.where(oh, gs, 0.0), axis=1, keepdims=True)
    wh = h2 * (ge / gs_n)
    oht = (lax.broadcasted_iota(jnp.int32, (G, N), 0) == br_ref[...]).astype(_f32)
    pooled = jnp.dot(oht, wh, preferred_element_type=_f32)
    out_ref[...] = jnp.dot(pooled, lw_ref[...],
                           preferred_element_type=_f32) + lb_ref[...]


_final = pl.pallas_call(
    _final_body,
    out_shape=jax.ShapeDtypeStruct((G, H), _f32),
)


def kernel(x, edge_index, batch, W1, a_src1, a_dst1, b1, W2, a_src2, a_dst2,
           b2, gate_W, gate_b, lin_W, lin_b):
    x_pad = jnp.pad(x, ((0, N_PAD - N), (0, 0)))
    loop = jnp.arange(N, dtype=jnp.int32)
    pad_idx = jnp.full((E_PAD - E - N,), N, jnp.int32)
    src = jnp.concatenate([edge_index[0], loop, pad_idx]).reshape(NW, NCH, CH)
    dst = jnp.concatenate([edge_index[1], loop, pad_idx]).reshape(NW, NCH, CH)

    h1, as1, ad1 = _proj(x_pad, W1, a_src1.reshape(H, 1), a_dst1.reshape(H, 1))
    acc1, sp1 = _edge_kernel(src, dst, h1, as1.reshape(NROW, 128),
                             ad1.reshape(NROW, 128))
    s1 = _spsum(sp1).reshape(N_PAD, 1)
    h2, as2, ad2 = _combine(acc1, s1, b1.reshape(1, H),
                            W2, a_src2.reshape(H, 1), a_dst2.reshape(H, 1))
    acc2, sp2 = _edge_kernel(src, dst, h2, as2.reshape(NROW, 128),
                             ad2.reshape(NROW, 128))
    s2 = _spsum(sp2).reshape(N_PAD, 1)
    return _final(acc2, s2, b2.reshape(1, H),
                  batch.reshape(N, 1), batch.reshape(1, N),
                  gate_W, gate_b.reshape(1, 1), lin_W, lin_b.reshape(1, H))
